# Initial kernel scaffold; baseline (speedup 1.0000x reference)
import jax
import jax.numpy as jnp
from jax.experimental import pallas as pl


def kernel(feat, edge_index, W, attn_l, attn_r, bias):
    raise NotImplementedError("write your pallas kernel here")


# speedup vs baseline: 8.0910x; 8.0910x over previous
import functools

import jax
import jax.numpy as jnp
from jax import lax
from jax.experimental import pallas as pl
from jax.experimental.pallas import tpu as pltpu
from jax.experimental.pallas import tpu_sc as plsc

NC, NS, L = 2, 16, 16
NW = NC * NS


def _proj_body(f_ref, w_ref, al_ref, ar_ref, sel_ref, h_ref, el_ref, er_ref):
    h = jnp.dot(f_ref[...], w_ref[...], preferred_element_type=jnp.float32)
    h_ref[...] = h
    el_ref[...] = jnp.dot(h * al_ref[...], sel_ref[...],
                          preferred_element_type=jnp.float32)
    er_ref[...] = jnp.dot(h * ar_ref[...], sel_ref[...],
                          preferred_element_type=jnp.float32)


def _project(feat, W, al, ar, sel, n, fin, HD):
    BN = 1000
    return pl.pallas_call(
        _proj_body,
        grid=(n // BN,),
        in_specs=[
            pl.BlockSpec((BN, fin), lambda i: (i, 0)),
            pl.BlockSpec((fin, HD), lambda i: (0, 0)),
            pl.BlockSpec((1, HD), lambda i: (0, 0)),
            pl.BlockSpec((1, HD), lambda i: (0, 0)),
            pl.BlockSpec((HD, L), lambda i: (0, 0)),
        ],
        out_specs=[
            pl.BlockSpec((BN, HD), lambda i: (i, 0)),
            pl.BlockSpec((BN, L), lambda i: (i, 0)),
            pl.BlockSpec((BN, L), lambda i: (i, 0)),
        ],
        out_shape=[
            jax.ShapeDtypeStruct((n, HD), jnp.float32),
            jax.ShapeDtypeStruct((n, L), jnp.float32),
            jax.ShapeDtypeStruct((n, L), jnp.float32),
        ],
    )(feat, W, al, ar, sel)


def _make_pass1(n, npad, e_cnt):
    ept = e_cnt // NW
    C1 = 1000
    nch = ept // C1
    rpt = npad // NS
    mesh = plsc.VectorSubcoreMesh(core_axis_name="c", subcore_axis_name="s")

    @functools.partial(
        pl.kernel,
        out_type=(
            jax.ShapeDtypeStruct((e_cnt, L), jnp.float32),
            jax.ShapeDtypeStruct((NC * npad, L), jnp.float32),
        ),
        mesh=mesh,
        compiler_params=pltpu.CompilerParams(use_tc_tiling_on_sc=False),
        scratch_types=[
            pltpu.VMEM((C1,), jnp.int32),
            pltpu.VMEM((C1,), jnp.int32),
            pltpu.VMEM((C1, L), jnp.float32),
            pltpu.VMEM((C1, L), jnp.float32),
            pltpu.VMEM((C1, L), jnp.float32),
            pltpu.VMEM_SHARED((npad, L), jnp.float32),
            pltpu.SemaphoreType.DMA,
            pltpu.SemaphoreType.DMA,
        ],
    )
    def pass1(src_hbm, dst_hbm, el_hbm, er_hbm, ex_hbm, dp_hbm,
              src_v, dst_v, elr, err, exr, dacc, sem1, sem2):
        c = lax.axis_index("c")
        s = lax.axis_index("s")
        wid = c * NS + s
        base = wid * ept

        def zrow(i, _):
            exr[i, :] = jnp.zeros((L,), jnp.float32)
            return 0
        lax.fori_loop(0, rpt, zrow, 0)
        pltpu.sync_copy(exr.at[pl.ds(0, rpt)], dacc.at[pl.ds(s * rpt, rpt)])
        plsc.subcore_barrier()

        def chunk(g, _):
            off = base + g * C1
            pltpu.sync_copy(src_hbm.at[pl.ds(off, C1)], src_v)
            pltpu.sync_copy(dst_hbm.at[pl.ds(off, C1)], dst_v)
            cp1 = pltpu.async_copy(el_hbm.at[src_v], elr, sem1)
            cp2 = pltpu.async_copy(er_hbm.at[dst_v], err, sem2)
            cp1.wait()
            cp2.wait()

            def ebody(i, _):
                v = elr[i, :] + err[i, :]
                sc = jnp.where(v >= 0.0, v, v * 0.2)
                exr[i, :] = jnp.exp(sc)
                return 0
            lax.fori_loop(0, C1, ebody, 0)

            pltpu.sync_copy(exr, dacc.at[dst_v], add=True)
            pltpu.sync_copy(exr, ex_hbm.at[pl.ds(off, C1)])
            return 0
        lax.fori_loop(0, nch, chunk, 0)

        plsc.subcore_barrier()
        pltpu.sync_copy(dacc.at[pl.ds(s * rpt, rpt)], exr.at[pl.ds(0, rpt)])
        pltpu.sync_copy(exr.at[pl.ds(0, rpt)],
                        dp_hbm.at[pl.ds(c * npad + s * rpt, rpt)])

    return pass1


def _make_rdenom(n, npad, H):
    def _rd_body(dp_ref, rd_ref):
        d = dp_ref[0:n, :] + dp_ref[npad:npad + n, :]
        rd_ref[...] = (1.0 / H) / jnp.where(d == 0.0, 1.0, d)

    def rdenom(dp):
        return pl.pallas_call(
            _rd_body,
            out_shape=jax.ShapeDtypeStruct((n, L), jnp.float32),
        )(dp)
    return rdenom


def _make_pass2(n, npad, e_cnt, H, D, HD):
    ept = e_cnt // NW
    C2 = 40
    nch = ept // C2
    rpt = npad // NS
    WB = C2
    nwb = rpt // WB
    mesh = plsc.VectorSubcoreMesh(core_axis_name="c", subcore_axis_name="s")

    @functools.partial(
        pl.kernel,
        out_type=jax.ShapeDtypeStruct((NC * npad, D), jnp.float32),
        mesh=mesh,
        compiler_params=pltpu.CompilerParams(use_tc_tiling_on_sc=False),
        scratch_types=[
            pltpu.VMEM((C2,), jnp.int32),
            pltpu.VMEM((C2,), jnp.int32),
            pltpu.VMEM((C2, HD), jnp.float32),
            pltpu.VMEM((C2, L), jnp.float32),
            pltpu.VMEM((C2, L), jnp.float32),
            pltpu.VMEM((C2, L), jnp.float32),
            pltpu.VMEM((C2, D), jnp.float32),
            pltpu.VMEM_SHARED((npad, D), jnp.float32),
            pltpu.SemaphoreType.DMA,
            pltpu.SemaphoreType.DMA,
        ],
    )
    def pass2(src_hbm, dst_hbm, ex_hbm, rd_hbm, h_hbm, op_hbm,
              src_v, dst_v, hrows, exr, rdr, wv, cbuf, oacc,
              sem1, sem2):
        c = lax.axis_index("c")
        s = lax.axis_index("s")
        wid = c * NS + s
        base = wid * ept

        def zrow(i, _):
            for j in range(D // L):
                cbuf[i, pl.ds(j * L, L)] = jnp.zeros((L,), jnp.float32)
            return 0
        lax.fori_loop(0, WB, zrow, 0)
        for t in range(nwb):
            pltpu.sync_copy(cbuf, oacc.at[pl.ds(s * rpt + t * WB, WB)])
        plsc.subcore_barrier()

        def chunk(g, _):
            off = base + g * C2
            pltpu.sync_copy(src_hbm.at[pl.ds(off, C2)], src_v)
            pltpu.sync_copy(dst_hbm.at[pl.ds(off, C2)], dst_v)
            cp1 = pltpu.async_copy(h_hbm.at[src_v], hrows, sem1)
            cp2 = pltpu.async_copy(rd_hbm.at[dst_v], rdr, sem2)
            pltpu.sync_copy(ex_hbm.at[pl.ds(off, C2)], exr)
            cp2.wait()

            def wbody(i, _):
                wv[i, :] = exr[i, :] * rdr[i, :]
                return 0
            lax.fori_loop(0, C2, wbody, 0)
            cp1.wait()

            dnums = lax.GatherDimensionNumbers(
                offset_dims=(), collapsed_slice_dims=(0,),
                start_index_map=(0,))

            def ebody(i, _):
                wvec = wv[i, :]
                accs = [None] * (D // L)
                for hh in range(H):
                    wsplat = lax.gather(
                        wvec, jnp.full((L, 1), hh, jnp.int32), dnums, (1,),
                        mode=lax.GatherScatterMode.PROMISE_IN_BOUNDS)
                    for j in range(D // L):
                        hv = hrows[i, pl.ds(hh * D + j * L, L)]
                        accs[j] = (wsplat * hv if hh == 0
                                   else accs[j] + wsplat * hv)
                for j in range(D // L):
                    cbuf[i, pl.ds(j * L, L)] = accs[j]
                return 0
            lax.fori_loop(0, C2, ebody, 0)

            pltpu.sync_copy(cbuf, oacc.at[dst_v], add=True)
            return 0
        lax.fori_loop(0, nch, chunk, 0)

        plsc.subcore_barrier()
        for t in range(nwb):
            pltpu.sync_copy(oacc.at[pl.ds(s * rpt + t * WB, WB)], cbuf)
            pltpu.sync_copy(cbuf,
                            op_hbm.at[pl.ds(c * npad + s * rpt + t * WB, WB)])

    return pass2


def _make_final(n, npad, H, D):
    def _fin_body(op_ref, b_ref, o_ref):
        bm = jnp.mean(b_ref[...], axis=0, keepdims=True)
        o_ref[...] = op_ref[0:n, :] + op_ref[npad:npad + n, :] + bm

    def final(op, b):
        return pl.pallas_call(
            _fin_body,
            out_shape=jax.ShapeDtypeStruct((n, D), jnp.float32),
        )(op, b)
    return final


def kernel(feat, edge_index, W, attn_l, attn_r, bias):
    n, fin = feat.shape
    e_cnt = edge_index.shape[1]
    H, D = attn_l.shape
    HD = H * D

    src = edge_index[0]
    dst = edge_index[1]
    al = attn_l.reshape(1, HD)
    ar = attn_r.reshape(1, HD)
    sel = jnp.pad(jnp.repeat(jnp.eye(H, dtype=jnp.float32), D, axis=0),
                  ((0, 0), (0, L - H)))

    npad = ((n + 2047) // 2048) * 2048

    h, el16, er16 = _project(feat, W, al, ar, sel, n, fin, HD)
    ex, dp = _make_pass1(n, npad, e_cnt)(src, dst, el16, er16)
    rd = _make_rdenom(n, npad, H)(dp)
    op = _make_pass2(n, npad, e_cnt, H, D, HD)(src, dst, ex, rd, h)
    return _make_final(n, npad, H, D)(op, bias.reshape(H, D))


# speedup vs baseline: 12.2053x; 1.5085x over previous
import functools

import jax
import jax.numpy as jnp
from jax import lax
from jax.experimental import pallas as pl
from jax.experimental.pallas import tpu as pltpu
from jax.experimental.pallas import tpu_sc as plsc

NC, NS, L = 2, 16, 16
NW = NC * NS


def _proj_body(f_ref, w_ref, al_ref, ar_ref, sel_ref, hb_ref, el_ref, er_ref):
    h = jnp.dot(f_ref[...], w_ref[...], preferred_element_type=jnp.float32)
    hb_ref[...] = h.astype(jnp.bfloat16)
    el_ref[...] = jnp.dot(h * al_ref[...], sel_ref[...],
                          preferred_element_type=jnp.float32)
    er_ref[...] = jnp.dot(h * ar_ref[...], sel_ref[...],
                          preferred_element_type=jnp.float32)


def _project(feat, W, al, ar, sel, n, fin, HD):
    BN = 2000
    return pl.pallas_call(
        _proj_body,
        grid=(n // BN,),
        in_specs=[
            pl.BlockSpec((BN, fin), lambda i: (i, 0)),
            pl.BlockSpec((fin, HD), lambda i: (0, 0)),
            pl.BlockSpec((1, HD), lambda i: (0, 0)),
            pl.BlockSpec((1, HD), lambda i: (0, 0)),
            pl.BlockSpec((HD, L), lambda i: (0, 0)),
        ],
        out_specs=[
            pl.BlockSpec((BN, HD), lambda i: (i, 0)),
            pl.BlockSpec((BN, L), lambda i: (i, 0)),
            pl.BlockSpec((BN, L), lambda i: (i, 0)),
        ],
        out_shape=[
            jax.ShapeDtypeStruct((n, HD), jnp.bfloat16),
            jax.ShapeDtypeStruct((n, L), jnp.float32),
            jax.ShapeDtypeStruct((n, L), jnp.float32),
        ],
    )(feat, W, al, ar, sel)


def _make_pass1(n, npad, e_cnt):
    ept = e_cnt // NW
    C1 = 1000
    nch = ept // C1
    rpt = npad // NS
    mesh = plsc.VectorSubcoreMesh(core_axis_name="c", subcore_axis_name="s")

    @functools.partial(
        pl.kernel,
        out_type=(
            jax.ShapeDtypeStruct((e_cnt, L), jnp.float32),
            jax.ShapeDtypeStruct((NC * npad, L), jnp.float32),
        ),
        mesh=mesh,
        compiler_params=pltpu.CompilerParams(use_tc_tiling_on_sc=False,
                                             needs_layout_passes=False),
        scratch_types=[
            pltpu.VMEM((C1,), jnp.int32),
            pltpu.VMEM((C1,), jnp.int32),
            pltpu.VMEM((C1, L), jnp.float32),
            pltpu.VMEM((C1, L), jnp.float32),
            pltpu.VMEM((C1, L), jnp.float32),
            pltpu.VMEM_SHARED((npad, L), jnp.float32),
            pltpu.SemaphoreType.DMA,
            pltpu.SemaphoreType.DMA,
        ],
    )
    def pass1(src_hbm, dst_hbm, el_hbm, er_hbm, ex_hbm, dp_hbm,
              src_v, dst_v, elr, err, exr, dacc, sem1, sem2):
        c = lax.axis_index("c")
        s = lax.axis_index("s")
        wid = c * NS + s
        base = wid * ept

        def zrow(i, _):
            exr[i, :] = jnp.zeros((L,), jnp.float32)
            return 0
        lax.fori_loop(0, rpt, zrow, 0)
        pltpu.sync_copy(exr.at[pl.ds(0, rpt)], dacc.at[pl.ds(s * rpt, rpt)])
        plsc.subcore_barrier()

        def chunk(g, _):
            off = base + g * C1
            pltpu.sync_copy(src_hbm.at[pl.ds(off, C1)], src_v)
            pltpu.sync_copy(dst_hbm.at[pl.ds(off, C1)], dst_v)
            cp1 = pltpu.async_copy(el_hbm.at[src_v], elr, sem1)
            cp2 = pltpu.async_copy(er_hbm.at[dst_v], err, sem2)
            cp1.wait()
            cp2.wait()

            def ebody(i, _):
                v = elr[i, :] + err[i, :]
                sc = jnp.where(v >= 0.0, v, v * 0.2)
                exr[i, :] = jnp.exp(sc)
                return 0
            lax.fori_loop(0, C1, ebody, 0)

            pltpu.sync_copy(exr, dacc.at[dst_v], add=True)
            pltpu.sync_copy(exr, ex_hbm.at[pl.ds(off, C1)])
            return 0
        lax.fori_loop(0, nch, chunk, 0)

        plsc.subcore_barrier()
        pltpu.sync_copy(dacc.at[pl.ds(s * rpt, rpt)], exr.at[pl.ds(0, rpt)])
        pltpu.sync_copy(exr.at[pl.ds(0, rpt)],
                        dp_hbm.at[pl.ds(c * npad + s * rpt, rpt)])

    return pass1


def _make_rdenom(n, npad, H):
    def _rd_body(dp_ref, rd_ref):
        d = dp_ref[0:n, :] + dp_ref[npad:npad + n, :]
        rd_ref[...] = (1.0 / H) / jnp.where(d == 0.0, 1.0, d)

    def rdenom(dp):
        return pl.pallas_call(
            _rd_body,
            out_shape=jax.ShapeDtypeStruct((n, L), jnp.float32),
        )(dp)
    return rdenom


def _make_pass2(n, npad, e_cnt, H, D, HD):
    ept = e_cnt // NW
    C2 = 40
    nch = ept // C2
    rpt = npad // NS
    WB = C2
    nwb = rpt // WB
    nblk = HD // (2 * L)
    mesh = plsc.VectorSubcoreMesh(core_axis_name="c", subcore_axis_name="s")

    @functools.partial(
        pl.kernel,
        out_type=jax.ShapeDtypeStruct((NC * npad, D), jnp.float32),
        mesh=mesh,
        compiler_params=pltpu.CompilerParams(use_tc_tiling_on_sc=False,
                                             needs_layout_passes=False),
        scratch_types=[
            [pltpu.VMEM((C2,), jnp.int32)] * 2,
            [pltpu.VMEM((C2,), jnp.int32)] * 2,
            [pltpu.VMEM((C2, HD), jnp.bfloat16)] * 2,
            [pltpu.VMEM((C2, L), jnp.float32)] * 2,
            [pltpu.VMEM((C2, L), jnp.float32)] * 2,
            pltpu.VMEM((C2, D), jnp.float32),
            pltpu.VMEM_SHARED((npad, D), jnp.float32),
            [pltpu.SemaphoreType.DMA] * 2,
            [pltpu.SemaphoreType.DMA] * 2,
            [pltpu.SemaphoreType.DMA] * 2,
        ],
    )
    def pass2(src_hbm, dst_hbm, ex_hbm, rd_hbm, h_hbm, op_hbm,
              src_v, dst_v, hrows, exr, rdr, cbuf, oacc, semh, semr, semx):
        c = lax.axis_index("c")
        s = lax.axis_index("s")
        wid = c * NS + s
        base = wid * ept

        def zrow(i, _):
            for j in range(D // L):
                cbuf[i, pl.ds(j * L, L)] = jnp.zeros((L,), jnp.float32)
            return 0
        lax.fori_loop(0, WB, zrow, 0)
        for t in range(nwb):
            pltpu.sync_copy(cbuf, oacc.at[pl.ds(s * rpt + t * WB, WB)])
        plsc.subcore_barrier()

        def issue(b, off):
            pltpu.sync_copy(src_hbm.at[pl.ds(off, C2)], src_v[b])
            pltpu.sync_copy(dst_hbm.at[pl.ds(off, C2)], dst_v[b])
            pltpu.async_copy(h_hbm.at[src_v[b]], hrows[b], semh[b])
            pltpu.async_copy(rd_hbm.at[dst_v[b]], rdr[b], semr[b])
            pltpu.async_copy(ex_hbm.at[pl.ds(off, C2)], exr[b], semx[b])

        def wait(b):
            pltpu.make_async_copy(h_hbm.at[src_v[b]], hrows[b], semh[b]).wait()
            pltpu.make_async_copy(rd_hbm.at[dst_v[b]], rdr[b], semr[b]).wait()
            pltpu.make_async_copy(ex_hbm.at[pl.ds(0, C2)], exr[b],
                                  semx[b]).wait()

        dnums = lax.GatherDimensionNumbers(
            offset_dims=(), collapsed_slice_dims=(0,), start_index_map=(0,))

        def compute(b):
            def ebody(i, _):
                wvec = exr[b][i, :] * rdr[b][i, :]
                wsp = [lax.gather(
                    wvec, jnp.full((L, 1), hh, jnp.int32), dnums, (1,),
                    mode=lax.GatherScatterMode.PROMISE_IN_BOUNDS)
                    for hh in range(H)]
                accs = [None] * (D // L)
                for m in range(nblk):
                    hh = m // (nblk // H)
                    d0 = (m % (nblk // H)) * 2 * L
                    j0, j1 = d0 // L, d0 // L + 1
                    hv = hrows[b][i, pl.ds(m * 2 * L, 2 * L)]
                    a0, a1 = plsc.unpack(hv, format=plsc.PackFormat.INTERLEAVED)
                    accs[j0] = (wsp[hh] * a0 if hh == 0
                                else accs[j0] + wsp[hh] * a0)
                    accs[j1] = (wsp[hh] * a1 if hh == 0
                                else accs[j1] + wsp[hh] * a1)
                for j in range(D // L):
                    cbuf[i, pl.ds(j * L, L)] = accs[j]
                return 0
            lax.fori_loop(0, C2, ebody, 0)
            pltpu.sync_copy(cbuf, oacc.at[dst_v[b]], add=True)

        issue(0, base)
        def super_chunk(q, _):
            g = q * 2
            wait(0)
            issue(1, base + (g + 1) * C2)
            compute(0)
            wait(1)

            @pl.when(g + 2 < nch)
            def _():
                issue(0, base + (g + 2) * C2)
            compute(1)
            return 0
        lax.fori_loop(0, nch // 2, super_chunk, 0)

        plsc.subcore_barrier()
        for t in range(nwb):
            pltpu.sync_copy(oacc.at[pl.ds(s * rpt + t * WB, WB)], cbuf)
            pltpu.sync_copy(cbuf,
                            op_hbm.at[pl.ds(c * npad + s * rpt + t * WB, WB)])

    return pass2


def _make_final(n, npad, H, D):
    def _fin_body(op_ref, b_ref, o_ref):
        bm = jnp.mean(b_ref[...], axis=0, keepdims=True)
        o_ref[...] = op_ref[0:n, :] + op_ref[npad:npad + n, :] + bm

    def final(op, b):
        return pl.pallas_call(
            _fin_body,
            out_shape=jax.ShapeDtypeStruct((n, D), jnp.float32),
        )(op, b)
    return final


def kernel(feat, edge_index, W, attn_l, attn_r, bias):
    n, fin = feat.shape
    e_cnt = edge_index.shape[1]
    H, D = attn_l.shape
    HD = H * D

    src = edge_index[0]
    dst = edge_index[1]
    col = jnp.arange(HD)
    orig = (col // 32) * 32 + (col % 32) // 2 + 16 * (col % 2)
    Wp = W[:, orig]
    al = attn_l.reshape(HD)[orig].reshape(1, HD)
    ar = attn_r.reshape(HD)[orig].reshape(1, HD)
    sel = jnp.pad(jnp.repeat(jnp.eye(H, dtype=jnp.float32), D, axis=0),
                  ((0, 0), (0, L - H)))

    npad = ((n + 2047) // 2048) * 2048

    hb, el16, er16 = _project(feat, Wp, al, ar, sel, n, fin, HD)
    ex, dp = _make_pass1(n, npad, e_cnt)(src, dst, el16, er16)
    rd = _make_rdenom(n, npad, H)(dp)
    op = _make_pass2(n, npad, e_cnt, H, D, HD)(src, dst, ex, rd, hb)
    return _make_final(n, npad, H, D)(op, bias.reshape(H, D))
